# baseline (device time: 66327 ns/iter reference)
import jax
import jax.numpy as jnp
from jax import lax
from jax.experimental import pallas as pl
from jax.experimental.pallas import tpu as pltpu

N_DEV = 4
MC = 256


def kernel(x, w_mat):
    m_per, k = x.shape
    _, n = w_mat.shape
    n_per = n // N_DEV
    m_tot = m_per * N_DEV
    n_xc = m_per // MC

    def body(x_hbm, w_hbm, out_hbm,
             xb_ref, xstage_ref, wstage_ref, wb_ref, y_ref, qblk_ref,
             a2a_ref, amax_ref, ostage_ref, x_sems, w_sems, o_sems,
             amax_send_sems, amax_recv_sems, blk_send_sems, blk_recv_sems):
        me = lax.axis_index("i")

        def x_dma(c):
            return pltpu.make_async_copy(
                x_hbm.at[pl.ds(c * MC, MC), :],
                xstage_ref.at[c % 2],
                x_sems.at[c % 2],
            )

        def w_dma(j):
            return pltpu.make_async_copy(
                w_hbm.at[:, pl.ds(j * n_per, n_per)],
                wstage_ref.at[j % 2],
                w_sems.at[j % 2],
            )

        x_dma(0).start()
        w_dma(0).start()

        barrier_sem = pltpu.get_barrier_semaphore()
        for d in range(1, N_DEV):
            pl.semaphore_signal(
                barrier_sem, inc=1,
                device_id=((me + d) % N_DEV,),
                device_id_type=pl.DeviceIdType.MESH,
            )
        pl.semaphore_wait(barrier_sem, N_DEV - 1)

        with jax.named_scope("xstage"):
            for c in range(n_xc):
                if c + 1 < n_xc:
                    x_dma(c + 1).start()
                x_dma(c).wait()
                xb_ref[c * MC:(c + 1) * MC, :] = (
                    xstage_ref[c % 2].astype(jnp.bfloat16))

        amax = jnp.float32(0.0)
        with jax.named_scope("gemm"):
            w_dma(0).wait()
            wb_ref[0] = wstage_ref[0].astype(jnp.bfloat16)
            w_dma(1).start()
            for j in range(N_DEV):
                if j + 1 < N_DEV:
                    w_dma(j + 1).wait()
                    if j + 2 < N_DEV:
                        w_dma(j + 2).start()
                    wb_ref[(j + 1) % 2] = (
                        wstage_ref[(j + 1) % 2].astype(jnp.bfloat16))
                yj = jnp.maximum(
                    jnp.dot(xb_ref[...], wb_ref[j % 2],
                            preferred_element_type=jnp.float32),
                    0.0)
                y_ref[:, j * n_per:(j + 1) * n_per] = yj
                amax = jnp.maximum(amax, jnp.max(yj))

        with jax.named_scope("amax_xchg"):
            amax_ref[pl.ds(me, 1)] = jnp.full((1, 8, 128), amax, jnp.float32)
            amax_sends = []
            for d in range(1, N_DEV):
                peer = (me + d) % N_DEV
                r = pltpu.make_async_remote_copy(
                    src_ref=amax_ref.at[me],
                    dst_ref=amax_ref.at[me],
                    send_sem=amax_send_sems.at[d],
                    recv_sem=amax_recv_sems.at[me],
                    device_id=(peer,),
                    device_id_type=pl.DeviceIdType.MESH,
                )
                r.start()
                amax_sends.append(r)
            for d in range(1, N_DEV):
                src = (me + d) % N_DEV
                rr = pltpu.make_async_remote_copy(
                    src_ref=amax_ref.at[src],
                    dst_ref=amax_ref.at[src],
                    send_sem=amax_send_sems.at[d],
                    recv_sem=amax_recv_sems.at[src],
                    device_id=(src,),
                    device_id_type=pl.DeviceIdType.MESH,
                )
                rr.wait_recv()
            for r in amax_sends:
                r.wait_send()

        amax_g = jnp.max(amax_ref[...])
        scale = amax_g / 448.0
        inv_scale = 448.0 / amax_g

        def quant(col):
            return jnp.minimum(
                y_ref[:, pl.ds(col * n_per, n_per)] * inv_scale, 448.0
            ).astype(jnp.float8_e4m3fn)

        blk_sends = []
        with jax.named_scope("quant_a2a_send"):
            for d in (2, 1, 3):
                peer = (me + d) % N_DEV
                qblk_ref[pl.ds(peer, 1)] = quant(peer)[None]
                r = pltpu.make_async_remote_copy(
                    src_ref=qblk_ref.at[peer],
                    dst_ref=a2a_ref.at[me],
                    send_sem=blk_send_sems.at[d],
                    recv_sem=blk_recv_sems.at[me],
                    device_id=(peer,),
                    device_id_type=pl.DeviceIdType.MESH,
                )
                r.start()
                blk_sends.append(r)

        out_dmas = []
        with jax.named_scope("own_store"):
            ostage_ref[0] = quant(me).astype(jnp.float32) * scale
            o0 = pltpu.make_async_copy(
                ostage_ref.at[0],
                out_hbm.at[pl.ds(me * m_per, m_per), :],
                o_sems.at[0],
            )
            o0.start()
            out_dmas.append(o0)

        with jax.named_scope("a2a_wait_store"):
            for i, d in enumerate((1, 3, 2)):
                src = (me + d) % N_DEV
                rr = pltpu.make_async_remote_copy(
                    src_ref=qblk_ref.at[src],
                    dst_ref=a2a_ref.at[src],
                    send_sem=blk_send_sems.at[d],
                    recv_sem=blk_recv_sems.at[src],
                    device_id=(src,),
                    device_id_type=pl.DeviceIdType.MESH,
                )
                rr.wait_recv()
                slot = (i + 1) % 2
                if i >= 1:
                    out_dmas[i - 1].wait()
                blk = a2a_ref[pl.ds(src, 1)]
                ostage_ref[slot] = blk[0].astype(jnp.float32) * scale
                od = pltpu.make_async_copy(
                    ostage_ref.at[slot],
                    out_hbm.at[pl.ds(src * m_per, m_per), :],
                    o_sems.at[slot],
                )
                od.start()
                out_dmas.append(od)
            for r in blk_sends:
                r.wait_send()
            for od in out_dmas[-2:]:
                od.wait()

    return pl.pallas_call(
        body,
        out_shape=jax.ShapeDtypeStruct((m_tot, n_per), jnp.float32),
        in_specs=[
            pl.BlockSpec(memory_space=pltpu.MemorySpace.HBM),
            pl.BlockSpec(memory_space=pltpu.MemorySpace.HBM),
        ],
        out_specs=pl.BlockSpec(memory_space=pltpu.MemorySpace.HBM),
        scratch_shapes=[
            pltpu.VMEM((m_per, k), jnp.bfloat16),
            pltpu.VMEM((2, MC, k), jnp.float32),
            pltpu.VMEM((2, k, n_per), jnp.float32),
            pltpu.VMEM((2, k, n_per), jnp.bfloat16),
            pltpu.VMEM((m_per, n), jnp.float32),
            pltpu.VMEM((N_DEV, m_per, n_per), jnp.float8_e4m3fn),
            pltpu.VMEM((N_DEV, m_per, n_per), jnp.float8_e4m3fn),
            pltpu.VMEM((N_DEV, 8, 128), jnp.float32),
            pltpu.VMEM((2, m_per, n_per), jnp.float32),
            pltpu.SemaphoreType.DMA((2,)),
            pltpu.SemaphoreType.DMA((2,)),
            pltpu.SemaphoreType.DMA((2,)),
            pltpu.SemaphoreType.DMA((N_DEV,)),
            pltpu.SemaphoreType.DMA((N_DEV,)),
            pltpu.SemaphoreType.DMA((N_DEV,)),
            pltpu.SemaphoreType.DMA((N_DEV,)),
        ],
        compiler_params=pltpu.CompilerParams(
            collective_id=0,
            vmem_limit_bytes=100 * 1024 * 1024,
        ),
    )(x, w_mat)


# device time: 52875 ns/iter; 1.2544x vs baseline; 1.2544x over previous
import jax
import jax.numpy as jnp
from jax import lax
from jax.experimental import pallas as pl
from jax.experimental.pallas import tpu as pltpu

N_DEV = 4
MC = 256


def kernel(x, w_mat):
    m_per, k = x.shape
    _, n = w_mat.shape
    n_per = n // N_DEV
    m_tot = m_per * N_DEV
    n_xc = m_per // MC

    def body(x_hbm, w_hbm, out_hbm,
             xb_ref, xstage_ref, wstage_ref, y_ref, qblk_ref, a2a_ref,
             amax_ref, ostage_ref, x_sems, w_sems, o_sems,
             amax_send_sems, amax_recv_sems, blk_send_sems, blk_recv_sems):
        me = lax.axis_index("i")

        def x_dma(c):
            return pltpu.make_async_copy(
                x_hbm.at[pl.ds(c * MC, MC), :],
                xstage_ref.at[c % 2],
                x_sems.at[c % 2],
            )

        def w_dma(j):
            return pltpu.make_async_copy(
                w_hbm.at[:, pl.ds(j * n_per, n_per)],
                wstage_ref.at[j % 2],
                w_sems.at[j % 2],
            )

        x_dma(0).start()
        w_dma(0).start()

        barrier_sem = pltpu.get_barrier_semaphore()
        for d in range(1, N_DEV):
            pl.semaphore_signal(
                barrier_sem, inc=1,
                device_id=((me + d) % N_DEV,),
                device_id_type=pl.DeviceIdType.MESH,
            )
        pl.semaphore_wait(barrier_sem, N_DEV - 1)

        with jax.named_scope("xstage"):
            for c in range(n_xc):
                if c + 1 < n_xc:
                    x_dma(c + 1).start()
                x_dma(c).wait()
                xb_ref[c * MC:(c + 1) * MC, :] = (
                    xstage_ref[c % 2].astype(jnp.bfloat16))

        amax = jnp.float32(0.0)
        with jax.named_scope("gemm"):
            for j in range(N_DEV):
                if j + 1 < N_DEV:
                    w_dma(j + 1).start()
                w_dma(j).wait()
                yj = jnp.maximum(
                    jnp.dot(xb_ref[...],
                            wstage_ref[j % 2].astype(jnp.bfloat16),
                            preferred_element_type=jnp.float32),
                    0.0)
                y_ref[:, j * n_per:(j + 1) * n_per] = yj
                amax = jnp.maximum(amax, jnp.max(yj))

        with jax.named_scope("amax_xchg"):
            amax_ref[pl.ds(me, 1)] = jnp.full((1, 8, 128), amax, jnp.float32)
            amax_sends = []
            for d in range(1, N_DEV):
                peer = (me + d) % N_DEV
                r = pltpu.make_async_remote_copy(
                    src_ref=amax_ref.at[me],
                    dst_ref=amax_ref.at[me],
                    send_sem=amax_send_sems.at[d],
                    recv_sem=amax_recv_sems.at[me],
                    device_id=(peer,),
                    device_id_type=pl.DeviceIdType.MESH,
                )
                r.start()
                amax_sends.append(r)
            for d in range(1, N_DEV):
                src = (me + d) % N_DEV
                rr = pltpu.make_async_remote_copy(
                    src_ref=amax_ref.at[src],
                    dst_ref=amax_ref.at[src],
                    send_sem=amax_send_sems.at[d],
                    recv_sem=amax_recv_sems.at[src],
                    device_id=(src,),
                    device_id_type=pl.DeviceIdType.MESH,
                )
                rr.wait_recv()
            for r in amax_sends:
                r.wait_send()

        amax_g = jnp.max(amax_ref[...])
        scale = amax_g / 448.0
        inv_scale = 448.0 / amax_g

        def quant(col):
            return jnp.minimum(
                y_ref[:, pl.ds(col * n_per, n_per)] * inv_scale, 448.0
            ).astype(jnp.float8_e4m3fn)

        blk_sends = []
        with jax.named_scope("quant_a2a_send"):
            for d in (2, 1, 3):
                peer = (me + d) % N_DEV
                qblk_ref[pl.ds(peer, 1)] = quant(peer)[None]
                r = pltpu.make_async_remote_copy(
                    src_ref=qblk_ref.at[peer],
                    dst_ref=a2a_ref.at[me],
                    send_sem=blk_send_sems.at[d],
                    recv_sem=blk_recv_sems.at[me],
                    device_id=(peer,),
                    device_id_type=pl.DeviceIdType.MESH,
                )
                r.start()
                blk_sends.append(r)

        out_dmas = []
        with jax.named_scope("own_store"):
            ostage_ref[0] = (
                quant(me).astype(jnp.float32) * scale
            ).astype(jnp.bfloat16)
            o0 = pltpu.make_async_copy(
                ostage_ref.at[0],
                out_hbm.at[pl.ds(me * m_per, m_per), :],
                o_sems.at[0],
            )
            o0.start()
            out_dmas.append(o0)

        with jax.named_scope("a2a_wait_store"):
            for i, d in enumerate((1, 3, 2)):
                src = (me + d) % N_DEV
                rr = pltpu.make_async_remote_copy(
                    src_ref=qblk_ref.at[src],
                    dst_ref=a2a_ref.at[src],
                    send_sem=blk_send_sems.at[d],
                    recv_sem=blk_recv_sems.at[src],
                    device_id=(src,),
                    device_id_type=pl.DeviceIdType.MESH,
                )
                rr.wait_recv()
                blk = a2a_ref[pl.ds(src, 1)]
                ostage_ref[i + 1] = (
                    blk[0].astype(jnp.float32) * scale
                ).astype(jnp.bfloat16)
                od = pltpu.make_async_copy(
                    ostage_ref.at[i + 1],
                    out_hbm.at[pl.ds(src * m_per, m_per), :],
                    o_sems.at[i + 1],
                )
                od.start()
                out_dmas.append(od)
            for r in blk_sends:
                r.wait_send()
            for od in out_dmas:
                od.wait()

    return pl.pallas_call(
        body,
        out_shape=jax.ShapeDtypeStruct((m_tot, n_per), jnp.bfloat16),
        in_specs=[
            pl.BlockSpec(memory_space=pltpu.MemorySpace.HBM),
            pl.BlockSpec(memory_space=pltpu.MemorySpace.HBM),
        ],
        out_specs=pl.BlockSpec(memory_space=pltpu.MemorySpace.HBM),
        scratch_shapes=[
            pltpu.VMEM((m_per, k), jnp.bfloat16),
            pltpu.VMEM((2, MC, k), jnp.float32),
            pltpu.VMEM((2, k, n_per), jnp.float32),
            pltpu.VMEM((m_per, n), jnp.float32),
            pltpu.VMEM((N_DEV, m_per, n_per), jnp.float8_e4m3fn),
            pltpu.VMEM((N_DEV, m_per, n_per), jnp.float8_e4m3fn),
            pltpu.VMEM((N_DEV, 8, 128), jnp.float32),
            pltpu.VMEM((N_DEV, m_per, n_per), jnp.bfloat16),
            pltpu.SemaphoreType.DMA((2,)),
            pltpu.SemaphoreType.DMA((2,)),
            pltpu.SemaphoreType.DMA((N_DEV,)),
            pltpu.SemaphoreType.DMA((N_DEV,)),
            pltpu.SemaphoreType.DMA((N_DEV,)),
            pltpu.SemaphoreType.DMA((N_DEV,)),
            pltpu.SemaphoreType.DMA((N_DEV,)),
        ],
        compiler_params=pltpu.CompilerParams(
            collective_id=0,
            vmem_limit_bytes=100 * 1024 * 1024,
        ),
    )(x, w_mat)


# device time: 52299 ns/iter; 1.2682x vs baseline; 1.0110x over previous
import jax
import jax.numpy as jnp
from jax import lax
from jax.experimental import pallas as pl
from jax.experimental.pallas import tpu as pltpu

N_DEV = 4
MC = 256


def kernel(x, w_mat):
    m_per, k = x.shape
    _, n = w_mat.shape
    n_per = n // N_DEV
    m_tot = m_per * N_DEV
    n_xc = m_per // MC

    def body(x_hbm, w_hbm, out_hbm,
             xb_ref, xstage_ref, wstage_ref, y_ref, qblk_ref, a2a_ref,
             amax_ref, ostage_ref, x_sems, w_sems, o_sems,
             amax_send_sems, amax_recv_sems, blk_send_sems, blk_recv_sems):
        me = lax.axis_index("i")

        def x_dma(c):
            return pltpu.make_async_copy(
                x_hbm.at[pl.ds(c * MC, MC), :],
                xstage_ref.at[c % 2],
                x_sems.at[c % 2],
            )

        def w_dma(j):
            return pltpu.make_async_copy(
                w_hbm.at[:, pl.ds(j * n_per, n_per)],
                wstage_ref.at[j % 2],
                w_sems.at[j % 2],
            )

        x_dma(0).start()
        w_dma(0).start()

        barrier_sem = pltpu.get_barrier_semaphore()
        for d in range(1, N_DEV):
            pl.semaphore_signal(
                barrier_sem, inc=1,
                device_id=((me + d) % N_DEV,),
                device_id_type=pl.DeviceIdType.MESH,
            )
        pl.semaphore_wait(barrier_sem, N_DEV - 1)

        with jax.named_scope("xstage"):
            for c in range(n_xc):
                if c + 1 < n_xc:
                    x_dma(c + 1).start()
                x_dma(c).wait()
                xb_ref[c * MC:(c + 1) * MC, :] = (
                    xstage_ref[c % 2].astype(jnp.bfloat16))

        amax = jnp.float32(0.0)
        with jax.named_scope("gemm"):
            for j in range(N_DEV):
                if j + 1 < N_DEV:
                    w_dma(j + 1).start()
                w_dma(j).wait()
                yj = jnp.maximum(
                    jnp.dot(xb_ref[...],
                            wstage_ref[j % 2].astype(jnp.bfloat16),
                            preferred_element_type=jnp.float32),
                    0.0)
                y_ref[:, j * n_per:(j + 1) * n_per] = yj
                amax = jnp.maximum(amax, jnp.max(yj))

        with jax.named_scope("amax_xchg"):
            amax_ref[pl.ds(me, 1)] = jnp.full((1, 8, 128), amax, jnp.float32)
            amax_sends = []
            for d in range(1, N_DEV):
                peer = (me + d) % N_DEV
                r = pltpu.make_async_remote_copy(
                    src_ref=amax_ref.at[me],
                    dst_ref=amax_ref.at[me],
                    send_sem=amax_send_sems.at[d],
                    recv_sem=amax_recv_sems.at[me],
                    device_id=(peer,),
                    device_id_type=pl.DeviceIdType.MESH,
                )
                r.start()
                amax_sends.append(r)
            for d in range(1, N_DEV):
                src = (me + d) % N_DEV
                rr = pltpu.make_async_remote_copy(
                    src_ref=amax_ref.at[src],
                    dst_ref=amax_ref.at[src],
                    send_sem=amax_send_sems.at[d],
                    recv_sem=amax_recv_sems.at[src],
                    device_id=(src,),
                    device_id_type=pl.DeviceIdType.MESH,
                )
                rr.wait_recv()
            for r in amax_sends:
                r.wait_send()

        amax_g = jnp.max(amax_ref[...])
        scale = amax_g / 448.0
        inv_scale = 448.0 / amax_g

        mh = m_per // 2

        def quant_half(col, h):
            return jnp.minimum(
                y_ref[pl.ds(h * mh, mh), pl.ds(col * n_per, n_per)]
                * inv_scale, 448.0
            ).astype(jnp.float8_e4m3fn)

        blk_sends = []
        with jax.named_scope("quant_a2a_send"):
            for h in (0, 1):
                for d in (2, 1, 3):
                    peer = (me + d) % N_DEV
                    qblk_ref[pl.ds(peer, 1), pl.ds(h * mh, mh)] = (
                        quant_half(peer, h)[None])
                    r = pltpu.make_async_remote_copy(
                        src_ref=qblk_ref.at[peer, pl.ds(h * mh, mh)],
                        dst_ref=a2a_ref.at[me, pl.ds(h * mh, mh)],
                        send_sem=blk_send_sems.at[d, h],
                        recv_sem=blk_recv_sems.at[me, h],
                        device_id=(peer,),
                        device_id_type=pl.DeviceIdType.MESH,
                    )
                    r.start()
                    blk_sends.append(r)

        out_dmas = []
        with jax.named_scope("own_store"):
            for h in (0, 1):
                ostage_ref[0, h * mh:(h + 1) * mh] = (
                    quant_half(me, h).astype(jnp.float32) * scale
                ).astype(jnp.bfloat16)
            o0 = pltpu.make_async_copy(
                ostage_ref.at[0],
                out_hbm.at[pl.ds(me * m_per, m_per), :],
                o_sems.at[0, 0],
            )
            o0.start()
            out_dmas.append(o0)

        with jax.named_scope("a2a_wait_store"):
            slot = {1: 1, 3: 2, 2: 3}
            for h, d in ((0, 1), (0, 3), (0, 2), (1, 1), (1, 3), (1, 2)):
                src = (me + d) % N_DEV
                rr = pltpu.make_async_remote_copy(
                    src_ref=qblk_ref.at[src, pl.ds(h * mh, mh)],
                    dst_ref=a2a_ref.at[src, pl.ds(h * mh, mh)],
                    send_sem=blk_send_sems.at[d, h],
                    recv_sem=blk_recv_sems.at[src, h],
                    device_id=(src,),
                    device_id_type=pl.DeviceIdType.MESH,
                )
                rr.wait_recv()
                blk = a2a_ref[pl.ds(src, 1), pl.ds(h * mh, mh)]
                ostage_ref[slot[d], h * mh:(h + 1) * mh] = (
                    blk[0].astype(jnp.float32) * scale
                ).astype(jnp.bfloat16)
                od = pltpu.make_async_copy(
                    ostage_ref.at[slot[d], pl.ds(h * mh, mh)],
                    out_hbm.at[pl.ds(src * m_per + h * mh, mh), :],
                    o_sems.at[slot[d], h],
                )
                od.start()
                out_dmas.append(od)
            for r in blk_sends:
                r.wait_send()
            for od in out_dmas:
                od.wait()

    return pl.pallas_call(
        body,
        out_shape=jax.ShapeDtypeStruct((m_tot, n_per), jnp.bfloat16),
        in_specs=[
            pl.BlockSpec(memory_space=pltpu.MemorySpace.HBM),
            pl.BlockSpec(memory_space=pltpu.MemorySpace.HBM),
        ],
        out_specs=pl.BlockSpec(memory_space=pltpu.MemorySpace.HBM),
        scratch_shapes=[
            pltpu.VMEM((m_per, k), jnp.bfloat16),
            pltpu.VMEM((2, MC, k), jnp.float32),
            pltpu.VMEM((2, k, n_per), jnp.float32),
            pltpu.VMEM((m_per, n), jnp.float32),
            pltpu.VMEM((N_DEV, m_per, n_per), jnp.float8_e4m3fn),
            pltpu.VMEM((N_DEV, m_per, n_per), jnp.float8_e4m3fn),
            pltpu.VMEM((N_DEV, 8, 128), jnp.float32),
            pltpu.VMEM((N_DEV, m_per, n_per), jnp.bfloat16),
            pltpu.SemaphoreType.DMA((2,)),
            pltpu.SemaphoreType.DMA((2,)),
            pltpu.SemaphoreType.DMA((N_DEV, 2)),
            pltpu.SemaphoreType.DMA((N_DEV,)),
            pltpu.SemaphoreType.DMA((N_DEV,)),
            pltpu.SemaphoreType.DMA((N_DEV, 2)),
            pltpu.SemaphoreType.DMA((N_DEV, 2)),
        ],
        compiler_params=pltpu.CompilerParams(
            collective_id=0,
            vmem_limit_bytes=100 * 1024 * 1024,
        ),
    )(x, w_mat)


# device time: 52167 ns/iter; 1.2714x vs baseline; 1.0025x over previous
import jax
import jax.numpy as jnp
from jax import lax
from jax.experimental import pallas as pl
from jax.experimental.pallas import tpu as pltpu

N_DEV = 4
MC = 256


def kernel(x, w_mat):
    m_per, k = x.shape
    _, n = w_mat.shape
    n_per = n // N_DEV
    m_tot = m_per * N_DEV
    n_xc = m_per // MC

    def body(x_hbm, w_hbm, out_hbm,
             xb_ref, xstage_ref, wstage_ref, y_ref, qblk_ref, a2a_ref,
             amax_ref, ostage_ref, x_sems, w_sems, o_sems,
             amax_send_sems, amax_recv_sems, blk_send_sems, blk_recv_sems):
        me = lax.axis_index("i")

        def x_dma(c):
            return pltpu.make_async_copy(
                x_hbm.at[pl.ds(c * MC, MC), :],
                xstage_ref.at[c % 2],
                x_sems.at[c % 2],
            )

        def w_dma(j):
            return pltpu.make_async_copy(
                w_hbm.at[:, pl.ds(j * n_per, n_per)],
                wstage_ref.at[j % 2],
                w_sems.at[j % 2],
            )

        x_dma(0).start()
        w_dma(0).start()

        barrier_sem = pltpu.get_barrier_semaphore()
        for d in range(1, N_DEV):
            pl.semaphore_signal(
                barrier_sem, inc=1,
                device_id=((me + d) % N_DEV,),
                device_id_type=pl.DeviceIdType.MESH,
            )
        pl.semaphore_wait(barrier_sem, N_DEV - 1)

        with jax.named_scope("xstage"):
            for c in range(n_xc):
                if c + 1 < n_xc:
                    x_dma(c + 1).start()
                x_dma(c).wait()
                xb_ref[c * MC:(c + 1) * MC, :] = (
                    xstage_ref[c % 2].astype(jnp.bfloat16))

        amax = jnp.float32(0.0)
        with jax.named_scope("gemm"):
            for j in range(N_DEV):
                if j + 1 < N_DEV:
                    w_dma(j + 1).start()
                w_dma(j).wait()
                yj = jnp.dot(xb_ref[...],
                             wstage_ref[j % 2].astype(jnp.bfloat16),
                             preferred_element_type=jnp.float32)
                y_ref[:, j * n_per:(j + 1) * n_per] = yj
                amax = jnp.maximum(amax, jnp.max(yj))

        with jax.named_scope("amax_xchg"):
            amax_ref[pl.ds(me, 1)] = jnp.full((1, 8, 128), amax, jnp.float32)
            amax_sends = []
            for d in range(1, N_DEV):
                peer = (me + d) % N_DEV
                r = pltpu.make_async_remote_copy(
                    src_ref=amax_ref.at[me],
                    dst_ref=amax_ref.at[me],
                    send_sem=amax_send_sems.at[d],
                    recv_sem=amax_recv_sems.at[me],
                    device_id=(peer,),
                    device_id_type=pl.DeviceIdType.MESH,
                )
                r.start()
                amax_sends.append(r)
            for d in range(1, N_DEV):
                src = (me + d) % N_DEV
                rr = pltpu.make_async_remote_copy(
                    src_ref=amax_ref.at[src],
                    dst_ref=amax_ref.at[src],
                    send_sem=amax_send_sems.at[d],
                    recv_sem=amax_recv_sems.at[src],
                    device_id=(src,),
                    device_id_type=pl.DeviceIdType.MESH,
                )
                rr.wait_recv()
            for r in amax_sends:
                r.wait_send()

        amax_g = jnp.max(amax_ref[...])
        scale = amax_g / 448.0
        inv_scale = 448.0 / amax_g

        mh = m_per // 2

        def quant_half(col, h):
            return jnp.clip(
                y_ref[pl.ds(h * mh, mh), pl.ds(col * n_per, n_per)]
                * inv_scale, 0.0, 448.0
            ).astype(jnp.float8_e4m3fn)

        blk_sends = []
        with jax.named_scope("quant_a2a_send"):
            for h in (0, 1):
                for d in (2, 1, 3):
                    peer = (me + d) % N_DEV
                    qblk_ref[pl.ds(peer, 1), pl.ds(h * mh, mh)] = (
                        quant_half(peer, h)[None])
                    r = pltpu.make_async_remote_copy(
                        src_ref=qblk_ref.at[peer, pl.ds(h * mh, mh)],
                        dst_ref=a2a_ref.at[me, pl.ds(h * mh, mh)],
                        send_sem=blk_send_sems.at[d, h],
                        recv_sem=blk_recv_sems.at[me, h],
                        device_id=(peer,),
                        device_id_type=pl.DeviceIdType.MESH,
                    )
                    r.start()
                    blk_sends.append(r)

        out_dmas = []
        with jax.named_scope("own_store"):
            for h in (0, 1):
                ostage_ref[0, h * mh:(h + 1) * mh] = (
                    quant_half(me, h).astype(jnp.float32) * scale
                ).astype(jnp.bfloat16)
            o0 = pltpu.make_async_copy(
                ostage_ref.at[0],
                out_hbm.at[pl.ds(me * m_per, m_per), :],
                o_sems.at[0, 0],
            )
            o0.start()
            out_dmas.append(o0)

        with jax.named_scope("a2a_wait_store"):
            slot = {1: 1, 3: 2, 2: 3}
            for h, d in ((0, 1), (0, 3), (0, 2), (1, 1), (1, 3), (1, 2)):
                src = (me + d) % N_DEV
                rr = pltpu.make_async_remote_copy(
                    src_ref=qblk_ref.at[src, pl.ds(h * mh, mh)],
                    dst_ref=a2a_ref.at[src, pl.ds(h * mh, mh)],
                    send_sem=blk_send_sems.at[d, h],
                    recv_sem=blk_recv_sems.at[src, h],
                    device_id=(src,),
                    device_id_type=pl.DeviceIdType.MESH,
                )
                rr.wait_recv()
                blk = a2a_ref[pl.ds(src, 1), pl.ds(h * mh, mh)]
                ostage_ref[slot[d], h * mh:(h + 1) * mh] = (
                    blk[0].astype(jnp.float32) * scale
                ).astype(jnp.bfloat16)
                od = pltpu.make_async_copy(
                    ostage_ref.at[slot[d], pl.ds(h * mh, mh)],
                    out_hbm.at[pl.ds(src * m_per + h * mh, mh), :],
                    o_sems.at[slot[d], h],
                )
                od.start()
                out_dmas.append(od)
            for r in blk_sends:
                r.wait_send()
            for od in out_dmas:
                od.wait()

    return pl.pallas_call(
        body,
        out_shape=jax.ShapeDtypeStruct((m_tot, n_per), jnp.bfloat16),
        in_specs=[
            pl.BlockSpec(memory_space=pltpu.MemorySpace.HBM),
            pl.BlockSpec(memory_space=pltpu.MemorySpace.HBM),
        ],
        out_specs=pl.BlockSpec(memory_space=pltpu.MemorySpace.HBM),
        scratch_shapes=[
            pltpu.VMEM((m_per, k), jnp.bfloat16),
            pltpu.VMEM((2, MC, k), jnp.float32),
            pltpu.VMEM((2, k, n_per), jnp.float32),
            pltpu.VMEM((m_per, n), jnp.float32),
            pltpu.VMEM((N_DEV, m_per, n_per), jnp.float8_e4m3fn),
            pltpu.VMEM((N_DEV, m_per, n_per), jnp.float8_e4m3fn),
            pltpu.VMEM((N_DEV, 8, 128), jnp.float32),
            pltpu.VMEM((N_DEV, m_per, n_per), jnp.bfloat16),
            pltpu.SemaphoreType.DMA((2,)),
            pltpu.SemaphoreType.DMA((2,)),
            pltpu.SemaphoreType.DMA((N_DEV, 2)),
            pltpu.SemaphoreType.DMA((N_DEV,)),
            pltpu.SemaphoreType.DMA((N_DEV,)),
            pltpu.SemaphoreType.DMA((N_DEV, 2)),
            pltpu.SemaphoreType.DMA((N_DEV, 2)),
        ],
        compiler_params=pltpu.CompilerParams(
            collective_id=0,
            vmem_limit_bytes=100 * 1024 * 1024,
        ),
    )(x, w_mat)
